# baseline (speedup 1.0000x reference)
import functools

import jax
import jax.numpy as jnp
from jax import lax
from jax.experimental import pallas as pl
from jax.experimental.pallas import tpu as pltpu
from jax.experimental.pallas import tpu_sc as plsc


def _proj_body(tblT_ref, w_ref, bias_ref, out_ref):
    r = lax.dot_general(
        w_ref[...], tblT_ref[...],
        (((1,), (0,)), ((), ())),
        preferred_element_type=jnp.float32,
    )
    out_ref[...] = r[0] + bias_ref[0, 0]


def _project_table(tableT, w_scaled, bias_scaled, blk=65536):
    D, V = tableT.shape
    grid = pl.cdiv(V, blk)
    return pl.pallas_call(
        _proj_body,
        grid=(grid,),
        in_specs=[
            pl.BlockSpec((D, blk), lambda i: (0, i)),
            pl.BlockSpec((1, D), lambda i: (0, 0)),
            pl.BlockSpec(memory_space=pltpu.SMEM),
        ],
        out_specs=pl.BlockSpec((blk,), lambda i: (i,)),
        out_shape=jax.ShapeDtypeStruct((V,), jnp.float32),
    )(tableT, w_scaled, bias_scaled)


def _make_sc_gather(V, S, B, NC, NS, C):
    NW = NC * NS
    per_w = B // NW
    n_chunks = per_w // C
    n_seg = C // 16

    mesh = plsc.VectorSubcoreMesh(core_axis_name="c", subcore_axis_name="s")

    @functools.partial(
        pl.kernel,
        mesh=mesh,
        out_type=jax.ShapeDtypeStruct((B,), jnp.float32),
        scratch_types=[
            pltpu.VMEM((S * C,), jnp.int32),
            pltpu.VMEM((S * C,), jnp.int32),
            pltpu.VMEM((S * C,), jnp.float32),
            pltpu.VMEM((S * C,), jnp.float32),
            pltpu.VMEM((C,), jnp.float32),
            pltpu.SemaphoreType.DMA,
            pltpu.SemaphoreType.DMA,
            pltpu.SemaphoreType.DMA,
            pltpu.SemaphoreType.DMA,
        ],
    )
    def sc_gather(v_hbm, xp_hbm, out_hbm, idx0_v, idx1_v, vals0_v, vals1_v,
                  outb_v, ssem0, ssem1, gsem0, gsem1):
        wid = lax.axis_index("s") * NC + lax.axis_index("c")
        idxs = (idx0_v, idx1_v)
        vals = (vals0_v, vals1_v)
        ssems = (ssem0, ssem1)
        gsems = (gsem0, gsem1)

        def stage_start(ci, buf):
            base = wid * per_w + ci * C
            return pltpu.async_copy(
                xp_hbm.at[pl.ds(base * S, S * C)], idxs[buf], ssems[buf])

        def gather_start(buf):
            return pltpu.async_copy(
                v_hbm.at[idxs[buf]], vals[buf], gsems[buf])

        stage_start(0, 0).wait()
        s_next = stage_start(1, 1) if n_chunks > 1 else None
        g_cur = gather_start(0)

        for ci in range(n_chunks):
            buf = ci % 2
            nbuf = 1 - buf
            if ci + 1 < n_chunks:
                s_next.wait()
                g_next = gather_start(nbuf)
            g_cur.wait()
            if ci + 2 < n_chunks:
                s_next = stage_start(ci + 2, buf)

            def s_body(si, acc, _vr=vals[buf]):
                return tuple(
                    acc[jj] + _vr[pl.ds(si * C + jj * 16, 16)]
                    for jj in range(n_seg)
                )

            acc0 = tuple(jnp.zeros((16,), jnp.float32) for _ in range(n_seg))
            acc = lax.fori_loop(0, S, s_body, acc0)

            for jj in range(n_seg):
                z = acc[jj]
                outb_v[pl.ds(jj * 16, 16)] = 1.0 / (1.0 + jnp.exp(-z))
            base = wid * per_w + ci * C
            pltpu.sync_copy(outb_v, out_hbm.at[pl.ds(base, C)])
            if ci + 1 < n_chunks:
                g_cur = g_next

    return sc_gather


def kernel(x, table, W, b):
    B, S = x.shape
    V, D = table.shape

    x = x.astype(jnp.int32)

    info = plsc.get_sparse_core_info()
    NC, NS = info.num_cores, info.num_subcores
    NW = NC * NS
    C = 128
    n_chunks = B // (NW * C)
    xp = x.reshape(NW, n_chunks, C, S).swapaxes(2, 3).reshape(B * S)

    w_scaled = (W * (1.0 / S)).astype(jnp.float32)
    bias_scaled = (b * (1.0 / S)).reshape(1, 1).astype(jnp.float32)

    tT = jnp.swapaxes(table, 0, 1)
    v = _project_table(tT, w_scaled, bias_scaled)

    sc_gather = _make_sc_gather(V, S, B, NC, NS, C)
    return sc_gather(v, xp)


# speedup vs baseline: 1.5210x; 1.5210x over previous
import functools

import jax
import jax.numpy as jnp
from jax import lax
from jax.experimental import pallas as pl
from jax.experimental.pallas import tpu as pltpu
from jax.experimental.pallas import tpu_sc as plsc


def _proj_body(tblT_ref, w_ref, bias_ref, out_ref):
    r = lax.dot_general(
        w_ref[...], tblT_ref[...],
        (((1,), (0,)), ((), ())),
        preferred_element_type=jnp.float32,
    )
    out_ref[...] = r[0] + bias_ref[0, 0]


def _project_table(tableT, w_scaled, bias_scaled, blk=65536):
    D, V = tableT.shape
    grid = pl.cdiv(V, blk)
    return pl.pallas_call(
        _proj_body,
        grid=(grid,),
        in_specs=[
            pl.BlockSpec((D, blk), lambda i: (0, i)),
            pl.BlockSpec((1, D), lambda i: (0, 0)),
            pl.BlockSpec(memory_space=pltpu.SMEM),
        ],
        out_specs=pl.BlockSpec((blk,), lambda i: (i,)),
        out_shape=jax.ShapeDtypeStruct((V,), jnp.float32),
    )(tableT, w_scaled, bias_scaled)


def _make_sc_gather(V, S, B, NC, NS, C):
    NW = NC * NS
    per_w = B // NW
    n_chunks = per_w // C
    n_seg = C // 16

    mesh = plsc.VectorSubcoreMesh(core_axis_name="c", subcore_axis_name="s")

    @functools.partial(
        pl.kernel,
        mesh=mesh,
        out_type=jax.ShapeDtypeStruct((B,), jnp.float32),
        scratch_types=[
            pltpu.VMEM((S * C,), jnp.int32),
            pltpu.VMEM((S * C,), jnp.int32),
            pltpu.VMEM((S * C,), jnp.float32),
            pltpu.VMEM((S * C,), jnp.float32),
            pltpu.VMEM((C,), jnp.float32),
            pltpu.VMEM_SHARED((V,), jnp.float32),
            pltpu.SemaphoreType.DMA,
            pltpu.SemaphoreType.DMA,
            pltpu.SemaphoreType.DMA,
            pltpu.SemaphoreType.DMA,
        ],
    )
    def sc_gather(v_hbm, xp_hbm, out_hbm, idx0_v, idx1_v, vals0_v, vals1_v,
                  outb_v, v_sh, ssem0, ssem1, gsem0, gsem1):
        sid = lax.axis_index("s")
        wid = sid * NC + lax.axis_index("c")
        idxs = (idx0_v, idx1_v)
        vals = (vals0_v, vals1_v)
        ssems = (ssem0, ssem1)
        gsems = (gsem0, gsem1)

        def stage_start(ci, buf):
            base = wid * per_w + ci * C
            return pltpu.async_copy(
                xp_hbm.at[pl.ds(base * S, S * C)], idxs[buf], ssems[buf])

        def gather_start(buf):
            return pltpu.async_copy(
                v_sh.at[idxs[buf]], vals[buf], gsems[buf])

        s_first = stage_start(0, 0)

        @pl.when(sid == 0)
        def _copy_v():
            pltpu.sync_copy(v_hbm, v_sh)

        plsc.subcore_barrier()

        s_first.wait()
        s_next = stage_start(1, 1) if n_chunks > 1 else None
        g_cur = gather_start(0)

        for ci in range(n_chunks):
            buf = ci % 2
            nbuf = 1 - buf
            if ci + 1 < n_chunks:
                s_next.wait()
                g_next = gather_start(nbuf)
            g_cur.wait()
            if ci + 2 < n_chunks:
                s_next = stage_start(ci + 2, buf)

            def s_body(si, acc, _vr=vals[buf]):
                return tuple(
                    acc[jj] + _vr[pl.ds(si * C + jj * 16, 16)]
                    for jj in range(n_seg)
                )

            acc0 = tuple(jnp.zeros((16,), jnp.float32) for _ in range(n_seg))
            acc = lax.fori_loop(0, S, s_body, acc0)

            for jj in range(n_seg):
                z = acc[jj]
                outb_v[pl.ds(jj * 16, 16)] = 1.0 / (1.0 + jnp.exp(-z))
            base = wid * per_w + ci * C
            pltpu.sync_copy(outb_v, out_hbm.at[pl.ds(base, C)])
            if ci + 1 < n_chunks:
                g_cur = g_next

    return sc_gather


def kernel(x, table, W, b):
    B, S = x.shape
    V, D = table.shape

    x = x.astype(jnp.int32)

    info = plsc.get_sparse_core_info()
    NC, NS = info.num_cores, info.num_subcores
    NW = NC * NS
    C = 64
    n_chunks = B // (NW * C)
    xp = x.reshape(NW, n_chunks, C, S).swapaxes(2, 3).reshape(B * S)

    w_scaled = (W * (1.0 / S)).astype(jnp.float32)
    bias_scaled = (b * (1.0 / S)).reshape(1, 1).astype(jnp.float32)

    tT = jnp.swapaxes(table, 0, 1)
    v = _project_table(tT, w_scaled, bias_scaled)

    sc_gather = _make_sc_gather(V, S, B, NC, NS, C)
    return sc_gather(v, xp)
